# baseline (speedup 1.0000x reference)
import functools

import jax
import jax.numpy as jnp
from jax import lax
from jax.experimental import pallas as pl
from jax.experimental.pallas import tpu as pltpu
from jax.experimental.pallas import tpu_sc as plsc

N = 10000
E = 320000
D = 128
G = 64

NC = 2
NS = 16
NW = NC * NS

CHUNK = 128
CPT = 80
EPT = CPT * CHUNK
E_PAD = NW * EPT
IB = 8
NB = CPT // IB
N_PAD = 10240
ROWS_PT = N_PAD // NS
ZCHUNKS = ROWS_PT // CHUNK


def _sc_agg_body(h_hbm, idx_hbm, zeros_hbm, out_hbm,
                 idx_v, rows_v, agg_sh):
    c = lax.axis_index("c")
    s = lax.axis_index("s")
    wid = c * NS + s

    pltpu.sync_copy(zeros_hbm, rows_v)
    for k in range(ZCHUNKS):
        pltpu.sync_copy(rows_v,
                        agg_sh.at[pl.ds(s * ROWS_PT + k * CHUNK, CHUNK)])
    pltpu.sync_copy(idx_hbm.at[wid], idx_v)
    plsc.subcore_barrier()

    def body(j, carry):
        pltpu.sync_copy(h_hbm.at[idx_v.at[j, 0]], rows_v)
        pltpu.sync_copy(rows_v, agg_sh.at[idx_v.at[j, 1]], add=True)
        return carry

    lax.fori_loop(0, CPT, body, 0)
    plsc.subcore_barrier()

    pltpu.sync_copy(agg_sh.at[pl.ds(s * ROWS_PT, ROWS_PT)],
                    out_hbm.at[c, pl.ds(s * ROWS_PT, ROWS_PT)])


_sc_agg = pl.kernel(
    _sc_agg_body,
    out_type=jax.ShapeDtypeStruct((NC, N_PAD, D), jnp.float32),
    mesh=plsc.VectorSubcoreMesh(core_axis_name="c", subcore_axis_name="s"),
    scratch_types=[
        pltpu.VMEM((CPT, 2, CHUNK), jnp.int32),
        pltpu.VMEM((CHUNK, D), jnp.float32),
        pltpu.VMEM_SHARED((N_PAD, D), jnp.float32),
    ],
)


def _mlp_bn(h_ref, agg_ref, W1_ref, b1_ref, g_ref, be_ref, W2_ref, b2_ref):
    hsum = h_ref[...] + agg_ref[0, :N, :] + agg_ref[1, :N, :]
    z = jnp.dot(hsum, W1_ref[...], preferred_element_type=jnp.float32)
    z = jnp.maximum(z + b1_ref[...], 0.0)
    mu = jnp.mean(z, axis=0, keepdims=True)
    var = jnp.mean((z - mu) ** 2, axis=0, keepdims=True)
    zn = (z - mu) / jnp.sqrt(var + 1e-5) * g_ref[...] + be_ref[...]
    h2 = jnp.dot(zn, W2_ref[...], preferred_element_type=jnp.float32)
    return jnp.maximum(h2 + b2_ref[...], 0.0)


def _tc_layer_body(h_ref, agg_ref, W1_ref, b1_ref, g_ref, be_ref,
                   W2_ref, b2_ref, out_ref):
    out_ref[...] = _mlp_bn(h_ref, agg_ref, W1_ref, b1_ref, g_ref, be_ref,
                           W2_ref, b2_ref)


def _tc_layer_pool_body(h_ref, agg_ref, batch_ref, W1_ref, b1_ref, g_ref,
                        be_ref, W2_ref, b2_ref, out_ref):
    h3 = _mlp_bn(h_ref, agg_ref, W1_ref, b1_ref, g_ref, be_ref,
                 W2_ref, b2_ref)
    gids = lax.broadcasted_iota(jnp.int32, (G, N), 0)
    onehot = (gids == batch_ref[...]).astype(jnp.float32)
    out_ref[...] = jnp.dot(onehot, h3, preferred_element_type=jnp.float32)


_tc_layer = pl.pallas_call(
    _tc_layer_body,
    out_shape=jax.ShapeDtypeStruct((N, D), jnp.float32),
)

_tc_layer_pool = pl.pallas_call(
    _tc_layer_pool_body,
    out_shape=jax.ShapeDtypeStruct((G, D), jnp.float32),
)


def kernel(x, edge_index, batch, W1_0, b1_0, g_0, be_0, W2_0, b2_0,
           W1_1, b1_1, g_1, be_1, W2_1, b2_1,
           W1_2, b1_2, g_2, be_2, W2_2, b2_2):
    src, dst = lax.sort_key_val(edge_index[0], edge_index[1])
    ept_real = E // NW
    padt = EPT - ept_real
    junk = (N + jnp.arange(padt, dtype=jnp.int32) % (N_PAD - N))[None, :]
    src3 = jnp.pad(src.reshape(NW, ept_real), ((0, 0), (0, padt)),
                   mode="edge").reshape(NW, CPT, CHUNK)
    dst3 = jnp.concatenate(
        [dst.reshape(NW, ept_real),
         jnp.broadcast_to(junk, (NW, padt))], axis=1).reshape(NW, CPT, CHUNK)
    idx4 = jnp.stack([src3, dst3], axis=2)
    zeros = jnp.zeros((CHUNK, D), jnp.float32)
    batch2 = batch.reshape(1, N)

    params = [
        (W1_0, b1_0, g_0, be_0, W2_0, b2_0),
        (W1_1, b1_1, g_1, be_1, W2_1, b2_1),
        (W1_2, b1_2, g_2, be_2, W2_2, b2_2),
    ]
    h = x
    for i, (W1, b1, g, be, W2, b2) in enumerate(params):
        agg = _sc_agg(h, idx4, zeros)
        args = (h, agg, W1, b1.reshape(1, D), g.reshape(1, D),
                be.reshape(1, D), W2, b2.reshape(1, D))
        if i < 2:
            h = _tc_layer(*args)
        else:
            h = _tc_layer_pool(h, agg, batch2, *args[2:])
    return h


# speedup vs baseline: 1.3641x; 1.3641x over previous
import functools

import jax
import jax.numpy as jnp
from jax import lax
from jax.experimental import pallas as pl
from jax.experimental.pallas import tpu as pltpu
from jax.experimental.pallas import tpu_sc as plsc

N = 10000
E = 320000
D = 128
G = 64

NC = 2
NS = 16
NW = NC * NS

CHUNK = 128
CPT = 80
EPT = CPT * CHUNK
E_PAD = NW * EPT
N_PAD = 10240
ROWS_PT = N_PAD // NS
ZCHUNKS = ROWS_PT // CHUNK
L = 16


def _sc_agg_body(h_hbm, idx_hbm, zeros_hbm, out_hbm,
                 pk_v, iu_v, rows0_v, rows1_v, sem0, sem1, agg_sh):
    c = lax.axis_index("c")
    s = lax.axis_index("s")
    wid = c * NS + s

    pltpu.sync_copy(zeros_hbm, rows0_v)
    for k in range(ZCHUNKS):
        pltpu.sync_copy(rows0_v,
                        agg_sh.at[pl.ds(s * ROWS_PT + k * CHUNK, CHUNK)])
    pltpu.sync_copy(idx_hbm.at[wid], pk_v)
    plsc.subcore_barrier()

    rows = (rows0_v, rows1_v)
    sems = (sem0, sem1)

    def _unpack(j, u):
        for t in range(CHUNK // L):
            v = pk_v[j, pl.ds(t * L, L)]
            iu_v[u, 0, pl.ds(t * L, L)] = v & 0xFFFF
            iu_v[u, 1, pl.ds(t * L, L)] = lax.shift_right_logical(v, 16)

    def _gather(u, r):
        return pltpu.make_async_copy(h_hbm.at[iu_v.at[u, 0]],
                                     rows[r], sems[r])

    def _scatter(u, r):
        pltpu.sync_copy(rows[r], agg_sh.at[iu_v.at[u, 1]], add=True)

    _unpack(0, 0)
    _gather(0, 0).start()

    def body(i, carry):
        j0 = 2 * i
        _unpack(j0 + 1, 1)
        _gather(1, 1).start()
        _gather(0, 0).wait()
        _scatter(0, 0)

        @pl.when(i < CPT // 2 - 1)
        def _():
            _unpack(j0 + 2, 0)
            _gather(0, 0).start()

        _gather(1, 1).wait()
        _scatter(1, 1)
        return carry

    lax.fori_loop(0, CPT // 2, body, 0)
    plsc.subcore_barrier()

    pltpu.sync_copy(agg_sh.at[pl.ds(s * ROWS_PT, ROWS_PT)],
                    out_hbm.at[c, pl.ds(s * ROWS_PT, ROWS_PT)])


_sc_agg = pl.kernel(
    _sc_agg_body,
    out_type=jax.ShapeDtypeStruct((NC, N_PAD, D), jnp.float32),
    mesh=plsc.VectorSubcoreMesh(core_axis_name="c", subcore_axis_name="s"),
    scratch_types=[
        pltpu.VMEM((CPT, CHUNK), jnp.int32),
        pltpu.VMEM((2, 2, CHUNK), jnp.int32),
        pltpu.VMEM((CHUNK, D), jnp.float32),
        pltpu.VMEM((CHUNK, D), jnp.float32),
        pltpu.SemaphoreType.DMA,
        pltpu.SemaphoreType.DMA,
        pltpu.VMEM_SHARED((N_PAD, D), jnp.float32),
    ],
)


def _mlp_bn(h_ref, agg_ref, W1_ref, b1_ref, g_ref, be_ref, W2_ref, b2_ref):
    hsum = h_ref[...] + agg_ref[0, :N, :] + agg_ref[1, :N, :]
    z = jnp.dot(hsum, W1_ref[...], preferred_element_type=jnp.float32)
    z = jnp.maximum(z + b1_ref[...], 0.0)
    mu = jnp.mean(z, axis=0, keepdims=True)
    var = jnp.mean((z - mu) ** 2, axis=0, keepdims=True)
    zn = (z - mu) / jnp.sqrt(var + 1e-5) * g_ref[...] + be_ref[...]
    h2 = jnp.dot(zn, W2_ref[...], preferred_element_type=jnp.float32)
    return jnp.maximum(h2 + b2_ref[...], 0.0)


def _tc_layer_body(h_ref, agg_ref, W1_ref, b1_ref, g_ref, be_ref,
                   W2_ref, b2_ref, out_ref):
    out_ref[...] = _mlp_bn(h_ref, agg_ref, W1_ref, b1_ref, g_ref, be_ref,
                           W2_ref, b2_ref)


def _tc_layer_pool_body(h_ref, agg_ref, batch_ref, W1_ref, b1_ref, g_ref,
                        be_ref, W2_ref, b2_ref, out_ref):
    h3 = _mlp_bn(h_ref, agg_ref, W1_ref, b1_ref, g_ref, be_ref,
                 W2_ref, b2_ref)
    gids = lax.broadcasted_iota(jnp.int32, (G, N), 0)
    onehot = (gids == batch_ref[...]).astype(jnp.float32)
    out_ref[...] = jnp.dot(onehot, h3, preferred_element_type=jnp.float32)


_tc_layer = pl.pallas_call(
    _tc_layer_body,
    out_shape=jax.ShapeDtypeStruct((N, D), jnp.float32),
)

_tc_layer_pool = pl.pallas_call(
    _tc_layer_pool_body,
    out_shape=jax.ShapeDtypeStruct((G, D), jnp.float32),
)


def kernel(x, edge_index, batch, W1_0, b1_0, g_0, be_0, W2_0, b2_0,
           W1_1, b1_1, g_1, be_1, W2_1, b2_1,
           W1_2, b1_2, g_2, be_2, W2_2, b2_2):
    src, dst = lax.sort_key_val(edge_index[0], edge_index[1])
    ept_real = E // NW
    padt = EPT - ept_real
    junk = (N + jnp.arange(padt, dtype=jnp.int32) % (N_PAD - N))[None, :]
    src3 = jnp.pad(src.reshape(NW, ept_real), ((0, 0), (0, padt)),
                   mode="edge").reshape(NW, CPT, CHUNK)
    dst3 = jnp.concatenate(
        [dst.reshape(NW, ept_real),
         jnp.broadcast_to(junk, (NW, padt))], axis=1).reshape(NW, CPT, CHUNK)
    idx_pk = (dst3 << 16) | src3
    zeros = jnp.zeros((CHUNK, D), jnp.float32)
    batch2 = batch.reshape(1, N)

    params = [
        (W1_0, b1_0, g_0, be_0, W2_0, b2_0),
        (W1_1, b1_1, g_1, be_1, W2_1, b2_1),
        (W1_2, b1_2, g_2, be_2, W2_2, b2_2),
    ]
    h = x
    for i, (W1, b1, g, be, W2, b2) in enumerate(params):
        agg = _sc_agg(h, idx_pk, zeros)
        args = (h, agg, W1, b1.reshape(1, D), g.reshape(1, D),
                be.reshape(1, D), W2, b2.reshape(1, D))
        if i < 2:
            h = _tc_layer(*args)
        else:
            h = _tc_layer_pool(h, agg, batch2, *args[2:])
    return h


# speedup vs baseline: 3.5141x; 2.5761x over previous
import functools

import jax
import jax.numpy as jnp
from jax import lax
from jax.experimental import pallas as pl
from jax.experimental.pallas import tpu as pltpu
from jax.experimental.pallas import tpu_sc as plsc

N = 10000
E = 320000
D = 128
G = 64

NC = 2
NS = 16
NW = NC * NS

CHUNK = 128
CPT = 80
EPT = CPT * CHUNK
E_PAD = NW * EPT
N_PAD = 10240
ROWS_PT = N_PAD // NS
ZCHUNKS = ROWS_PT // CHUNK
L = 16


def _sc_agg_body(h_hbm, idx_hbm, zeros_hbm, out_hbm,
                 pk_v, iu_v, rows0_v, rows1_v, sem0, sem1, agg_sh):
    c = lax.axis_index("c")
    s = lax.axis_index("s")
    wid = c * NS + s

    pltpu.sync_copy(zeros_hbm, rows0_v)
    for k in range(ZCHUNKS):
        pltpu.sync_copy(rows0_v,
                        agg_sh.at[pl.ds(s * ROWS_PT + k * CHUNK, CHUNK)])
    pltpu.sync_copy(idx_hbm.at[wid], pk_v)
    plsc.subcore_barrier()

    rows = (rows0_v, rows1_v)
    sems = (sem0, sem1)

    def _unpack(j, u):
        for t in range(CHUNK // L):
            v = pk_v[j, pl.ds(t * L, L)]
            iu_v[u, 0, pl.ds(t * L, L)] = v & 0xFFFF
            iu_v[u, 1, pl.ds(t * L, L)] = lax.shift_right_logical(v, 16)

    def _gather(u, r):
        return pltpu.make_async_copy(h_hbm.at[iu_v.at[u, 0]],
                                     rows[r], sems[r])

    def _scatter(u, r):
        pltpu.sync_copy(rows[r], agg_sh.at[iu_v.at[u, 1]], add=True)

    _unpack(0, 0)
    _gather(0, 0).start()

    def body(i, carry):
        j0 = 2 * i
        _unpack(j0 + 1, 1)
        _gather(1, 1).start()
        _gather(0, 0).wait()
        _scatter(0, 0)

        @pl.when(i < CPT // 2 - 1)
        def _():
            _unpack(j0 + 2, 0)
            _gather(0, 0).start()

        _gather(1, 1).wait()
        _scatter(1, 1)
        return carry

    lax.fori_loop(0, CPT // 2, body, 0)
    plsc.subcore_barrier()

    pltpu.sync_copy(agg_sh.at[pl.ds(s * ROWS_PT, ROWS_PT)],
                    out_hbm.at[c, pl.ds(s * ROWS_PT, ROWS_PT)])


_sc_agg = pl.kernel(
    _sc_agg_body,
    out_type=jax.ShapeDtypeStruct((NC, N_PAD, D), jnp.float32),
    mesh=plsc.VectorSubcoreMesh(core_axis_name="c", subcore_axis_name="s"),
    scratch_types=[
        pltpu.VMEM((CPT, CHUNK), jnp.int32),
        pltpu.VMEM((2, 2, CHUNK), jnp.int32),
        pltpu.VMEM((CHUNK, D), jnp.float32),
        pltpu.VMEM((CHUNK, D), jnp.float32),
        pltpu.SemaphoreType.DMA,
        pltpu.SemaphoreType.DMA,
        pltpu.VMEM_SHARED((N_PAD, D), jnp.float32),
    ],
)


def _mlp_bn(h_ref, agg_ref, W1_ref, b1_ref, g_ref, be_ref, W2_ref, b2_ref):
    hsum = h_ref[...] + agg_ref[0, :N, :] + agg_ref[1, :N, :]
    z = jnp.dot(hsum, W1_ref[...], preferred_element_type=jnp.float32)
    z = jnp.maximum(z + b1_ref[...], 0.0)
    mu = jnp.mean(z, axis=0, keepdims=True)
    var = jnp.mean((z - mu) ** 2, axis=0, keepdims=True)
    zn = (z - mu) / jnp.sqrt(var + 1e-5) * g_ref[...] + be_ref[...]
    h2 = jnp.dot(zn, W2_ref[...], preferred_element_type=jnp.float32)
    return jnp.maximum(h2 + b2_ref[...], 0.0)


def _tc_layer_body(h_ref, agg_ref, W1_ref, b1_ref, g_ref, be_ref,
                   W2_ref, b2_ref, out_ref):
    out_ref[...] = _mlp_bn(h_ref, agg_ref, W1_ref, b1_ref, g_ref, be_ref,
                           W2_ref, b2_ref)


def _tc_layer_pool_body(h_ref, agg_ref, batch_ref, W1_ref, b1_ref, g_ref,
                        be_ref, W2_ref, b2_ref, out_ref):
    h3 = _mlp_bn(h_ref, agg_ref, W1_ref, b1_ref, g_ref, be_ref,
                 W2_ref, b2_ref)
    gids = lax.broadcasted_iota(jnp.int32, (G, N), 0)
    onehot = (gids == batch_ref[...]).astype(jnp.float32)
    out_ref[...] = jnp.dot(onehot, h3, preferred_element_type=jnp.float32)


_tc_layer = pl.pallas_call(
    _tc_layer_body,
    out_shape=jax.ShapeDtypeStruct((N, D), jnp.float32),
)

_tc_layer_pool = pl.pallas_call(
    _tc_layer_pool_body,
    out_shape=jax.ShapeDtypeStruct((G, D), jnp.float32),
)


def kernel(x, edge_index, batch, W1_0, b1_0, g_0, be_0, W2_0, b2_0,
           W1_1, b1_1, g_1, be_1, W2_1, b2_1,
           W1_2, b1_2, g_2, be_2, W2_2, b2_2):
    src = edge_index[0]
    dst = edge_index[1]
    ept_real = E // NW
    padt = EPT - ept_real
    junk = (N + jnp.arange(padt, dtype=jnp.int32) % (N_PAD - N))[None, :]
    src3 = jnp.pad(src.reshape(NW, ept_real), ((0, 0), (0, padt)),
                   mode="edge").reshape(NW, CPT, CHUNK)
    dst3 = jnp.concatenate(
        [dst.reshape(NW, ept_real),
         jnp.broadcast_to(junk, (NW, padt))], axis=1).reshape(NW, CPT, CHUNK)
    idx_pk = (dst3 << 16) | src3
    zeros = jnp.zeros((CHUNK, D), jnp.float32)
    batch2 = batch.reshape(1, N)

    params = [
        (W1_0, b1_0, g_0, be_0, W2_0, b2_0),
        (W1_1, b1_1, g_1, be_1, W2_1, b2_1),
        (W1_2, b1_2, g_2, be_2, W2_2, b2_2),
    ]
    h = x
    for i, (W1, b1, g, be, W2, b2) in enumerate(params):
        agg = _sc_agg(h, idx_pk, zeros)
        args = (h, agg, W1, b1.reshape(1, D), g.reshape(1, D),
                be.reshape(1, D), W2, b2.reshape(1, D))
        if i < 2:
            h = _tc_layer(*args)
        else:
            h = _tc_layer_pool(h, agg, batch2, *args[2:])
    return h
